# Initial kernel scaffold; baseline (speedup 1.0000x reference)
import jax
import jax.numpy as jnp
from jax.experimental import pallas as pl


def kernel(input_ids, word_embeddings, position_embeddings, ln_scale, ln_bias):
    raise NotImplementedError("write your pallas kernel here")


# speedup vs baseline: 1.3280x; 1.3280x over previous
import functools

import jax
import jax.numpy as jnp
from jax import lax
from jax.experimental import pallas as pl
from jax.experimental.pallas import tpu as pltpu
from jax.experimental.pallas import tpu_sc as plsc

B, S, V, H = 4, 2048, 128100, 1024
NT = B * S
LN_EPS = 1e-07

_info = plsc.get_sparse_core_info()
NC, NS = _info.num_cores, _info.num_subcores
NW = NC * NS
T_PER_W = NT // NW
CHUNK = 64
N_CHUNKS = T_PER_W // CHUNK


def _sc_gather(idx_flat, table):
    mesh = plsc.VectorSubcoreMesh(core_axis_name="c", subcore_axis_name="s")

    @functools.partial(
        pl.kernel,
        mesh=mesh,
        out_type=jax.ShapeDtypeStruct((NT, H), jnp.float32),
        scratch_types=[
            pltpu.VMEM((CHUNK,), jnp.int32),
            pltpu.VMEM((CHUNK, H), jnp.float32),
            pltpu.SemaphoreType.DMA,
        ],
    )
    def k(idx_hbm, table_hbm, out_hbm, idx_v, rows_v, sem):
        wid = lax.axis_index("s") * NC + lax.axis_index("c")
        base = wid * T_PER_W
        for c in range(N_CHUNKS):
            off = base + c * CHUNK
            pltpu.sync_copy(idx_hbm.at[pl.ds(off, CHUNK)], idx_v)
            pltpu.async_copy(table_hbm.at[idx_v], rows_v, sem).wait()
            pltpu.sync_copy(rows_v, out_hbm.at[pl.ds(off, CHUNK)])

    return k(idx_flat, table)


ROWS_BLK = 256


def _tc_add_ln(gathered, pos, scale, bias):
    n_blocks = NT // ROWS_BLK
    pos_blocks_per_batch = S // ROWS_BLK

    def body(g_ref, p_ref, s_ref, b_ref, o_ref):
        x = g_ref[...] + p_ref[...]
        mean = jnp.mean(x, axis=-1, keepdims=True)
        var = jnp.mean(jnp.square(x - mean), axis=-1, keepdims=True)
        normed = (x - mean) * lax.rsqrt(var + LN_EPS)
        o_ref[...] = normed * s_ref[...] + b_ref[...]

    return pl.pallas_call(
        body,
        grid=(n_blocks,),
        in_specs=[
            pl.BlockSpec((ROWS_BLK, H), lambda i: (i, 0)),
            pl.BlockSpec((ROWS_BLK, H), lambda i: (i % pos_blocks_per_batch, 0)),
            pl.BlockSpec((1, H), lambda i: (0, 0)),
            pl.BlockSpec((1, H), lambda i: (0, 0)),
        ],
        out_specs=pl.BlockSpec((ROWS_BLK, H), lambda i: (i, 0)),
        out_shape=jax.ShapeDtypeStruct((NT, H), jnp.float32),
    )(gathered, pos, scale, bias)


def kernel(input_ids, word_embeddings, position_embeddings, ln_scale, ln_bias):
    idx_flat = input_ids.reshape(NT).astype(jnp.int32)
    gathered = _sc_gather(idx_flat, word_embeddings)
    out = _tc_add_ln(
        gathered,
        position_embeddings,
        ln_scale.reshape(1, H),
        ln_bias.reshape(1, H),
    )
    return out.reshape(B, S, H)


# speedup vs baseline: 1.3493x; 1.0160x over previous
import functools

import jax
import jax.numpy as jnp
from jax import lax
from jax.experimental import pallas as pl
from jax.experimental.pallas import tpu as pltpu
from jax.experimental.pallas import tpu_sc as plsc

B, S, V, H = 4, 2048, 128100, 1024
NT = B * S
LN_EPS = 1e-07

_info = plsc.get_sparse_core_info()
NC, NS = _info.num_cores, _info.num_subcores
NW = NC * NS
T_PER_W = NT // NW
CHUNK = 32
N_CHUNKS = T_PER_W // CHUNK
NBUF = 2


def _sc_gather(idx_grouped, table):
    mesh = plsc.VectorSubcoreMesh(core_axis_name="c", subcore_axis_name="s")

    @functools.partial(
        pl.kernel,
        mesh=mesh,
        out_type=jax.ShapeDtypeStruct((NT, H), jnp.float32),
        scratch_types=[
            pltpu.VMEM((N_CHUNKS, CHUNK), jnp.int32),
            pltpu.VMEM((NBUF, CHUNK, H), jnp.float32),
            pltpu.SemaphoreType.DMA,
            pltpu.SemaphoreType.DMA,
            pltpu.SemaphoreType.DMA,
            pltpu.SemaphoreType.DMA,
        ],
    )
    def k(idx_hbm, table_hbm, out_hbm, idx_v, rows_v, g0, g1, w0, w1):
        wid = lax.axis_index("s") * NC + lax.axis_index("c")
        base = wid * T_PER_W
        gsem = (g0, g1)
        wsem = (w0, w1)
        pltpu.sync_copy(idx_hbm.at[wid], idx_v)
        gh = [None] * N_CHUNKS
        wh = [None] * N_CHUNKS
        for c in range(N_CHUNKS):
            b = c % NBUF
            if c >= NBUF:
                wh[c - NBUF].wait()
            gh[c] = pltpu.async_copy(
                table_hbm.at[idx_v.at[c]], rows_v.at[b], gsem[b])
            if c >= 1:
                pb = (c - 1) % NBUF
                gh[c - 1].wait()
                wh[c - 1] = pltpu.async_copy(
                    rows_v.at[pb],
                    out_hbm.at[pl.ds(base + (c - 1) * CHUNK, CHUNK)],
                    wsem[pb])
        last = N_CHUNKS - 1
        gh[last].wait()
        wh[last] = pltpu.async_copy(
            rows_v.at[last % NBUF],
            out_hbm.at[pl.ds(base + last * CHUNK, CHUNK)],
            wsem[last % NBUF])
        wh[last - 1].wait()
        wh[last].wait()

    return k(idx_grouped, table)


ROWS_BLK = 256


def _tc_add_ln(gathered, pos, scale, bias):
    n_blocks = NT // ROWS_BLK
    pos_blocks_per_batch = S // ROWS_BLK

    def body(g_ref, p_ref, s_ref, b_ref, o_ref):
        x = g_ref[...] + p_ref[...]
        mean = jnp.mean(x, axis=-1, keepdims=True)
        var = jnp.mean(jnp.square(x - mean), axis=-1, keepdims=True)
        normed = (x - mean) * lax.rsqrt(var + LN_EPS)
        o_ref[...] = normed * s_ref[...] + b_ref[...]

    return pl.pallas_call(
        body,
        grid=(n_blocks,),
        in_specs=[
            pl.BlockSpec((ROWS_BLK, H), lambda i: (i, 0)),
            pl.BlockSpec((ROWS_BLK, H), lambda i: (i % pos_blocks_per_batch, 0)),
            pl.BlockSpec((1, H), lambda i: (0, 0)),
            pl.BlockSpec((1, H), lambda i: (0, 0)),
        ],
        out_specs=pl.BlockSpec((ROWS_BLK, H), lambda i: (i, 0)),
        out_shape=jax.ShapeDtypeStruct((NT, H), jnp.float32),
    )(gathered, pos, scale, bias)


def kernel(input_ids, word_embeddings, position_embeddings, ln_scale, ln_bias):
    idx_grouped = input_ids.reshape(NW, N_CHUNKS, CHUNK).astype(jnp.int32)
    gathered = _sc_gather(idx_grouped, word_embeddings)
    out = _tc_add_ln(
        gathered,
        position_embeddings,
        ln_scale.reshape(1, H),
        ln_bias.reshape(1, H),
    )
    return out.reshape(B, S, H)


# speedup vs baseline: 1.3764x; 1.0201x over previous
import functools

import jax
import jax.numpy as jnp
from jax import lax
from jax.experimental import pallas as pl
from jax.experimental.pallas import tpu as pltpu
from jax.experimental.pallas import tpu_sc as plsc

B, S, V, H = 4, 2048, 128100, 1024
NT = B * S
LN_EPS = 1e-07

_info = plsc.get_sparse_core_info()
NC, NS = _info.num_cores, _info.num_subcores
NW = NC * NS
T_PER_W = NT // NW
CHUNK = 32
N_CHUNKS = T_PER_W // CHUNK
NBUF = 2


def _sc_gather(idx_grouped, table):
    mesh = plsc.VectorSubcoreMesh(core_axis_name="c", subcore_axis_name="s")

    @functools.partial(
        pl.kernel,
        mesh=mesh,
        out_type=jax.ShapeDtypeStruct((NT, H), jnp.float32),
        scratch_types=[
            pltpu.VMEM((N_CHUNKS, CHUNK), jnp.int32),
            pltpu.VMEM((NBUF, CHUNK, H), jnp.float32),
            pltpu.SemaphoreType.DMA,
            pltpu.SemaphoreType.DMA,
            pltpu.SemaphoreType.DMA,
            pltpu.SemaphoreType.DMA,
        ],
    )
    def k(idx_hbm, table_hbm, out_hbm, idx_v, rows_v, g0, g1, w0, w1):
        wid = lax.axis_index("s") * NC + lax.axis_index("c")
        base = wid * T_PER_W
        gsem = (g0, g1)
        wsem = (w0, w1)
        pltpu.sync_copy(idx_hbm.at[wid], idx_v)
        gh = [None] * N_CHUNKS
        wh = [None] * N_CHUNKS
        for c in range(N_CHUNKS):
            b = c % NBUF
            if c >= NBUF:
                wh[c - NBUF].wait()
            gh[c] = pltpu.async_copy(
                table_hbm.at[idx_v.at[c]], rows_v.at[b], gsem[b])
            if c >= 1:
                pb = (c - 1) % NBUF
                gh[c - 1].wait()
                wh[c - 1] = pltpu.async_copy(
                    rows_v.at[pb],
                    out_hbm.at[pl.ds(base + (c - 1) * CHUNK, CHUNK)],
                    wsem[pb])
        last = N_CHUNKS - 1
        gh[last].wait()
        wh[last] = pltpu.async_copy(
            rows_v.at[last % NBUF],
            out_hbm.at[pl.ds(base + last * CHUNK, CHUNK)],
            wsem[last % NBUF])
        wh[last - 1].wait()
        wh[last].wait()

    return k(idx_grouped, table)


ROWS_BLK = 256


def _tc_add_ln(gathered, pos, scale, bias):
    pos_blocks_per_batch = S // ROWS_BLK

    def body(g_ref, p_ref, s_ref, b_ref, o_ref):
        x = g_ref[...] + p_ref[...]
        mean = jnp.mean(x, axis=-1, keepdims=True)
        var = jnp.mean(jnp.square(x - mean), axis=-1, keepdims=True)
        normed = (x - mean) * lax.rsqrt(var + LN_EPS)
        o_ref[...] = normed * s_ref[...] + b_ref[...]

    return pl.pallas_call(
        body,
        grid=(pos_blocks_per_batch, B),
        in_specs=[
            pl.BlockSpec((ROWS_BLK, H), lambda i, j: (j * pos_blocks_per_batch + i, 0)),
            pl.BlockSpec((ROWS_BLK, H), lambda i, j: (i, 0)),
            pl.BlockSpec((1, H), lambda i, j: (0, 0)),
            pl.BlockSpec((1, H), lambda i, j: (0, 0)),
        ],
        out_specs=pl.BlockSpec((ROWS_BLK, H), lambda i, j: (j * pos_blocks_per_batch + i, 0)),
        out_shape=jax.ShapeDtypeStruct((NT, H), jnp.float32),
    )(gathered, pos, scale, bias)


def kernel(input_ids, word_embeddings, position_embeddings, ln_scale, ln_bias):
    idx_grouped = input_ids.reshape(NW, N_CHUNKS, CHUNK).astype(jnp.int32)
    gathered = _sc_gather(idx_grouped, word_embeddings)
    out = _tc_add_ln(
        gathered,
        position_embeddings,
        ln_scale.reshape(1, H),
        ln_bias.reshape(1, H),
    )
    return out.reshape(B, S, H)
